# Initial kernel scaffold; baseline (speedup 1.0000x reference)
import jax
import jax.numpy as jnp
from jax.experimental import pallas as pl


def kernel(words, table):
    raise NotImplementedError("write your pallas kernel here")


# speedup vs baseline: 4.2514x; 4.2514x over previous
import functools

import jax
import jax.numpy as jnp
from jax.experimental import pallas as pl
from jax.experimental.pallas import tpu as pltpu


def _round_up(x, m):
    return -(-x // m) * m


def _gather_kernel(idx_ref, table_ref, out_ref, *, tm, unroll):
    base = pl.program_id(0) * tm

    @pl.loop(0, tm // unroll)
    def _(g):
        r0 = g * unroll
        for u in range(unroll):
            out_ref[r0 + u] = table_ref[idx_ref[base + r0 + u]]


def kernel(words, table):
    orig_shape = words.shape
    num_vocab, num_emb = table.shape

    idx = words.reshape(-1).astype(jnp.int32)
    n = int(idx.shape[0])

    tm = 1024
    unroll = 16
    n_pad = _round_up(n, tm)
    if n_pad != n:
        idx = jnp.pad(idx, (0, n_pad - n))
    n_tiles = n_pad // tm

    table3 = table.reshape(num_vocab, 1, num_emb)

    out = pl.pallas_call(
        functools.partial(_gather_kernel, tm=tm, unroll=unroll),
        out_shape=jax.ShapeDtypeStruct((n_pad, 1, num_emb), table.dtype),
        grid_spec=pltpu.PrefetchScalarGridSpec(
            num_scalar_prefetch=1,
            grid=(n_tiles,),
            in_specs=[pl.BlockSpec((num_vocab, 1, num_emb),
                                   lambda i, idx_ref: (0, 0, 0))],
            out_specs=pl.BlockSpec((tm, 1, num_emb),
                                   lambda i, idx_ref: (i, 0, 0)),
        ),
        compiler_params=pltpu.CompilerParams(
            dimension_semantics=("parallel",),
            vmem_limit_bytes=44 * 1024 * 1024,
        ),
    )(idx, table3)

    return out[:n].reshape(*orig_shape, num_emb)


# speedup vs baseline: 4.4295x; 1.0419x over previous
import functools

import jax
import jax.numpy as jnp
from jax.experimental import pallas as pl
from jax.experimental.pallas import tpu as pltpu


def _round_up(x, m):
    return -(-x // m) * m


def _gather_kernel(idx_ref, table_ref, out_ref, *, tm, unroll):
    del unroll
    base = pl.program_id(0) * tm
    for r in range(tm):
        out_ref[r] = table_ref[idx_ref[base + r]]


def kernel(words, table):
    orig_shape = words.shape
    num_vocab, num_emb = table.shape

    idx = words.reshape(-1).astype(jnp.int32)
    n = int(idx.shape[0])

    tm = 512
    unroll = 0
    n_pad = _round_up(n, tm)
    if n_pad != n:
        idx = jnp.pad(idx, (0, n_pad - n))
    n_tiles = n_pad // tm

    table3 = table.reshape(num_vocab, 1, num_emb)

    out = pl.pallas_call(
        functools.partial(_gather_kernel, tm=tm, unroll=unroll),
        out_shape=jax.ShapeDtypeStruct((n_pad, 1, num_emb), table.dtype),
        grid_spec=pltpu.PrefetchScalarGridSpec(
            num_scalar_prefetch=1,
            grid=(n_tiles,),
            in_specs=[pl.BlockSpec((num_vocab, 1, num_emb),
                                   lambda i, idx_ref: (0, 0, 0))],
            out_specs=pl.BlockSpec((tm, 1, num_emb),
                                   lambda i, idx_ref: (i, 0, 0)),
        ),
        compiler_params=pltpu.CompilerParams(
            dimension_semantics=("parallel",),
            vmem_limit_bytes=44 * 1024 * 1024,
        ),
    )(idx, table3)

    return out[:n].reshape(*orig_shape, num_emb)


# speedup vs baseline: 5.6691x; 1.2799x over previous
import functools

import jax
import jax.numpy as jnp
from jax.experimental import pallas as pl
from jax.experimental.pallas import tpu as pltpu


def _round_up(x, m):
    return -(-x // m) * m


def _gather_kernel(idx_ref, table_ref, out_ref, *, tm, unroll):
    del unroll
    base = pl.program_id(0) * tm
    for r in range(tm):
        out_ref[r] = table_ref[idx_ref[base + r]]


def kernel(words, table):
    orig_shape = words.shape
    num_vocab, num_emb = table.shape

    idx = words.reshape(-1).astype(jnp.int32)
    n = int(idx.shape[0])

    tm = 1024
    unroll = 0
    n_pad = _round_up(n, tm)
    if n_pad != n:
        idx = jnp.pad(idx, (0, n_pad - n))
    n_tiles = n_pad // tm

    table3 = table.reshape(num_vocab, 1, num_emb)

    out = pl.pallas_call(
        functools.partial(_gather_kernel, tm=tm, unroll=unroll),
        out_shape=jax.ShapeDtypeStruct((n_pad, 1, num_emb), table.dtype),
        grid_spec=pltpu.PrefetchScalarGridSpec(
            num_scalar_prefetch=1,
            grid=(n_tiles,),
            in_specs=[pl.BlockSpec((num_vocab, 1, num_emb),
                                   lambda i, idx_ref: (0, 0, 0))],
            out_specs=pl.BlockSpec((tm, 1, num_emb),
                                   lambda i, idx_ref: (i, 0, 0)),
        ),
        compiler_params=pltpu.CompilerParams(
            dimension_semantics=("parallel",),
            vmem_limit_bytes=44 * 1024 * 1024,
        ),
    )(idx, table3)

    return out[:n].reshape(*orig_shape, num_emb)


# speedup vs baseline: 5.6904x; 1.0037x over previous
import functools

import jax
import jax.numpy as jnp
from jax.experimental import pallas as pl
from jax.experimental.pallas import tpu as pltpu


def _round_up(x, m):
    return -(-x // m) * m


def _gather_kernel(idx_ref, table_ref, out_ref, *, tm, unroll):
    del unroll
    base = pl.program_id(0) * tm
    for r in range(tm):
        out_ref[r] = table_ref[idx_ref[base + r]]


def kernel(words, table):
    orig_shape = words.shape
    num_vocab, num_emb = table.shape

    idx = words.reshape(-1).astype(jnp.int32)
    n = int(idx.shape[0])

    tm = 2048
    unroll = 0
    n_pad = _round_up(n, tm)
    if n_pad != n:
        idx = jnp.pad(idx, (0, n_pad - n))
    n_tiles = n_pad // tm

    table3 = table.reshape(num_vocab, 1, num_emb)

    out = pl.pallas_call(
        functools.partial(_gather_kernel, tm=tm, unroll=unroll),
        out_shape=jax.ShapeDtypeStruct((n_pad, 1, num_emb), table.dtype),
        grid_spec=pltpu.PrefetchScalarGridSpec(
            num_scalar_prefetch=1,
            grid=(n_tiles,),
            in_specs=[pl.BlockSpec((num_vocab, 1, num_emb),
                                   lambda i, idx_ref: (0, 0, 0))],
            out_specs=pl.BlockSpec((tm, 1, num_emb),
                                   lambda i, idx_ref: (i, 0, 0)),
        ),
        compiler_params=pltpu.CompilerParams(
            dimension_semantics=("parallel",),
            vmem_limit_bytes=44 * 1024 * 1024,
        ),
    )(idx, table3)

    return out[:n].reshape(*orig_shape, num_emb)
